# Initial kernel scaffold; baseline (speedup 1.0000x reference)
import jax
import jax.numpy as jnp
from jax.experimental import pallas as pl


def kernel(data, params, rel_rec, rel_send):
    raise NotImplementedError("write your pallas kernel here")


# speedup vs baseline: 1.2449x; 1.2449x over previous
import jax
import jax.numpy as jnp
import numpy as np
from jax.experimental import pallas as pl

V = 25
E = V * (V - 1)
H = 128
K = 4
C = 3
T = 20
N = 32
N_IN_ENC = C * T
TAU = 0.5
PRED_STEPS = 10
NSTEP = T - 1

_BN_DIV = float(np.sqrt(1.0 + 1e-5))


def _elu(x):
    return jnp.where(x > 0, x, jnp.exp(x) - 1.0)


def _mlp(h, w1, b1, w2, b2, bn):
    h = _elu(jnp.dot(h, w1, preferred_element_type=jnp.float32) + b1)
    h = _elu(jnp.dot(h, w2, preferred_element_type=jnp.float32) + b2)
    return bn[0:1] * h / _BN_DIV + bn[1:2]


def _fwd_body(x_ref, s0_ref, s10_ref, u_ref, relcat_ref, relrt_ref,
              m1w1, m1b1, m1w2, m1b2, m1bn,
              m2w1, m2b1, m2w2, m2b2, m2bn,
              m3w1, m3b1, m3w2, m3b2, m3bn,
              m4w1, m4b1, m4w2, m4b2, m4bn,
              fow, fob,
              mg1w, mg1b, mg2wa, mg2ba, mg2wb, mg2bb, mg2wc, mg2bc,
              gwi, gbi, gwh,
              o1w, o1b, o2w, o2b, o3w, o3b,
              preds_ref, prob_ref):
    relcat = relcat_ref[...]
    relrt = relrt_ref[...]

    x = x_ref[0]
    h = _mlp(x, m1w1[...], m1b1[...], m1w2[...], m1b2[...], m1bn[...])
    rs = jnp.dot(relcat, h, preferred_element_type=jnp.float32)
    eh = jnp.concatenate([rs[:E], rs[E:]], axis=1)
    h = _mlp(eh, m2w1[...], m2b1[...], m2w2[...], m2b2[...], m2bn[...])
    h_skip = h
    inc = jnp.dot(relrt, h, preferred_element_type=jnp.float32) / V
    h = _mlp(inc, m3w1[...], m3b1[...], m3w2[...], m3b2[...], m3bn[...])
    rs = jnp.dot(relcat, h, preferred_element_type=jnp.float32)
    eh = jnp.concatenate([rs[:E], rs[E:], h_skip], axis=1)
    h = _mlp(eh, m4w1[...], m4b1[...], m4w2[...], m4b2[...], m4bn[...])
    logits = jnp.dot(h, fow[...], preferred_element_type=jnp.float32) + fob[...]

    prob_ref[0] = jax.nn.softmax(logits, axis=-1)

    u = u_ref[0]
    g = -jnp.log(1e-10 - jnp.log(u + 1e-10))
    y = (logits + g) / TAU
    y_soft = jax.nn.softmax(y, axis=-1)
    idx = jnp.zeros((E, 1), jnp.int32)
    best = y[:, 0:1]
    for j in range(1, K):
        c = y[:, j:j + 1] > best
        idx = jnp.where(c, j, idx)
        best = jnp.where(c, y[:, j:j + 1], best)
    lane = jax.lax.broadcasted_iota(jnp.int32, (E, K), 1)
    hard = (lane == idx).astype(jnp.float32)
    edges = (hard - y_soft) + y_soft
    masks = [edges[:, k:k + 1] for k in range(1, K)]

    w1cat = mg1w[...]
    b1cat = mg1b[...]
    w2s = (mg2wa[...], mg2wb[...], mg2wc[...])
    b2s = (mg2ba[...], mg2bb[...], mg2bc[...])
    wi = gwi[...]; bi = gbi[...]; wh = gwh[...]
    seq0 = s0_ref[0]
    seq10 = s10_ref[0]

    hidden = jnp.zeros((V, H), jnp.float32)
    pred = None
    for s in range(NSTEP):
        ins = seq0 if s == 0 else (seq10 if s == PRED_STEPS else pred)
        rs = jnp.dot(relcat, hidden, preferred_element_type=jnp.float32)
        pre = jnp.concatenate([rs[:E], rs[E:]], axis=1)
        m1 = jnp.tanh(jnp.dot(pre, w1cat, preferred_element_type=jnp.float32) + b1cat)
        msgs = jnp.zeros((E, H), jnp.float32)
        for k in range(3):
            mk = jnp.tanh(jnp.dot(m1[:, k * H:(k + 1) * H], w2s[k],
                                  preferred_element_type=jnp.float32) + b2s[k])
            msgs = msgs + (mk * masks[k]) / 3.0
        agg = jnp.dot(relrt, msgs, preferred_element_type=jnp.float32) / 3.0
        gin = jnp.dot(ins, wi, preferred_element_type=jnp.float32) + bi
        gh = jnp.dot(agg, wh, preferred_element_type=jnp.float32)
        r = jax.nn.sigmoid(gin[:, :H] + gh[:, :H])
        i = jax.nn.sigmoid(gin[:, H:2 * H] + gh[:, H:2 * H])
        nn = jnp.tanh(gin[:, 2 * H:] + r * gh[:, 2 * H:])
        hidden = (1.0 - i) * nn + i * hidden
        p = jax.nn.relu(jnp.dot(hidden, o1w[...], preferred_element_type=jnp.float32) + o1b[...])
        p = jax.nn.relu(jnp.dot(p, o2w[...], preferred_element_type=jnp.float32) + o2b[...])
        p = jnp.dot(p, o3w[...], preferred_element_type=jnp.float32) + o3b[...]
        pred = ins + p
        preds_ref[0, :, s, :] = pred


def _row(b):
    return b.reshape(1, -1)


def kernel(data, params, rel_rec, rel_send):
    p = params
    x = jnp.transpose(data, (0, 3, 1, 2)).reshape(N, V, N_IN_ENC)
    seq = jnp.transpose(data, (0, 2, 3, 1))
    s0 = seq[:, 0]
    s10 = seq[:, PRED_STEPS]
    u = jax.random.uniform(jax.random.key(42), (N, E, K), dtype=jnp.float32)
    relcat = jnp.concatenate([rel_rec, rel_send], axis=0)
    relrt = jnp.transpose(rel_rec)

    weights = []
    for name in ('mlp1', 'mlp2', 'mlp3', 'mlp4'):
        weights += [p[name + '_fc1_w'], _row(p[name + '_fc1_b']),
                    p[name + '_fc2_w'], _row(p[name + '_fc2_b']),
                    jnp.stack([p[name + '_bn_g'], p[name + '_bn_b']])]
    weights += [p['fc_out_w'], _row(p['fc_out_b'])]
    weights += [jnp.concatenate([p['msg1_%d_w' % k] for k in (1, 2, 3)], axis=1),
                jnp.concatenate([_row(p['msg1_%d_b' % k]) for k in (1, 2, 3)], axis=1)]
    for k in (1, 2, 3):
        weights += [p['msg2_%d_w' % k], _row(p['msg2_%d_b' % k])]
    weights += [jnp.concatenate([p['input_r_w'], p['input_i_w'], p['input_n_w']], axis=1),
                jnp.concatenate([_row(p['input_r_b']), _row(p['input_i_b']),
                                 _row(p['input_n_b'])], axis=1),
                jnp.concatenate([p['hidden_r_w'], p['hidden_i_w'], p['hidden_n_w']], axis=1)]
    weights += [p['out_fc1_w'], _row(p['out_fc1_b']),
                p['out_fc2_w'], _row(p['out_fc2_b']),
                p['out_fc3_w'], _row(p['out_fc3_b'])]

    def wspec(w):
        nd = w.ndim
        return pl.BlockSpec(w.shape, lambda i, _nd=nd: (0,) * _nd)

    in_specs = [
        pl.BlockSpec((1, V, N_IN_ENC), lambda i: (i, 0, 0)),
        pl.BlockSpec((1, V, C), lambda i: (i, 0, 0)),
        pl.BlockSpec((1, V, C), lambda i: (i, 0, 0)),
        pl.BlockSpec((1, E, K), lambda i: (i, 0, 0)),
        pl.BlockSpec((2 * E, V), lambda i: (0, 0)),
        pl.BlockSpec((V, E), lambda i: (0, 0)),
    ] + [wspec(w) for w in weights]

    out_specs = [
        pl.BlockSpec((1, V, NSTEP, C), lambda i: (i, 0, 0, 0)),
        pl.BlockSpec((1, E, K), lambda i: (i, 0, 0)),
    ]
    out_shape = [
        jax.ShapeDtypeStruct((N, V, NSTEP, C), jnp.float32),
        jax.ShapeDtypeStruct((N, E, K), jnp.float32),
    ]

    preds, prob = pl.pallas_call(
        _fwd_body,
        grid=(N,),
        in_specs=in_specs,
        out_specs=out_specs,
        out_shape=out_shape,
    )(x, s0, s10, u, relcat, relrt, *weights)
    return preds, prob


# speedup vs baseline: 1.8184x; 1.4607x over previous
import jax
import jax.numpy as jnp
import numpy as np
from jax.experimental import pallas as pl

V = 25
E = V * (V - 1)
H = 128
K = 4
C = 3
T = 20
N = 32
N_IN_ENC = C * T
TAU = 0.5
PRED_STEPS = 10
NSTEP = T - 1
B = 4
G = N // B

_BN_DIV = float(np.sqrt(1.0 + 1e-5))


def _elu(x):
    return jnp.where(x > 0, x, jnp.exp(x) - 1.0)


def _mlp(h, w1, b1, w2, b2, bn):
    h = _elu(jnp.dot(h, w1, preferred_element_type=jnp.float32) + b1)
    h = _elu(jnp.dot(h, w2, preferred_element_type=jnp.float32) + b2)
    return bn[0:1] * h / _BN_DIV + bn[1:2]


def _fwd_body(x_ref, s0_ref, s10_ref, u_ref, relcat_ref, relrt_ref,
              m1w1, m1b1, m1w2, m1b2, m1bn,
              m2w1, m2b1, m2w2, m2b2, m2bn,
              m3w1, m3b1, m3w2, m3b2, m3bn,
              m4w1, m4b1, m4w2, m4b2, m4bn,
              fow, fob,
              mg1w, mg1b, mg2wa, mg2ba, mg2wb, mg2bb, mg2wc, mg2bc,
              gwi, gbi, gwh,
              o1w, o1b, o2w, o2b, o3w, o3b,
              preds_ref, prob_ref):
    BE = B * E
    relcat = relcat_ref[...]
    relrt = relrt_ref[...]

    x = x_ref[0]
    h = _mlp(x, m1w1[...], m1b1[...], m1w2[...], m1b2[...], m1bn[...])
    rs = jnp.dot(relcat, h, preferred_element_type=jnp.float32)
    eh = jnp.concatenate([rs[:BE], rs[BE:]], axis=1)
    h = _mlp(eh, m2w1[...], m2b1[...], m2w2[...], m2b2[...], m2bn[...])
    h_skip = h
    inc = jnp.dot(relrt, h, preferred_element_type=jnp.float32) / V
    h = _mlp(inc, m3w1[...], m3b1[...], m3w2[...], m3b2[...], m3bn[...])
    rs = jnp.dot(relcat, h, preferred_element_type=jnp.float32)
    eh = jnp.concatenate([rs[:BE], rs[BE:], h_skip], axis=1)
    h = _mlp(eh, m4w1[...], m4b1[...], m4w2[...], m4b2[...], m4bn[...])
    logits = jnp.dot(h, fow[...], preferred_element_type=jnp.float32) + fob[...]

    prob_ref[0] = jax.nn.softmax(logits, axis=-1)

    u = u_ref[0]
    g = -jnp.log(1e-10 - jnp.log(u + 1e-10))
    y = (logits + g) / TAU
    y_soft = jax.nn.softmax(y, axis=-1)
    idx = jnp.zeros((BE, 1), jnp.int32)
    best = y[:, 0:1]
    for j in range(1, K):
        c = y[:, j:j + 1] > best
        idx = jnp.where(c, j, idx)
        best = jnp.where(c, y[:, j:j + 1], best)
    lane = jax.lax.broadcasted_iota(jnp.int32, (BE, K), 1)
    hard = (lane == idx).astype(jnp.float32)
    edges = (hard - y_soft) + y_soft
    masks = [edges[:, k:k + 1] for k in range(1, K)]

    w1cat = mg1w[...]
    b1cat = mg1b[...]
    w2s = (mg2wa[...], mg2wb[...], mg2wc[...])
    b2s = (mg2ba[...], mg2bb[...], mg2bc[...])
    wi = gwi[...]; bi = gbi[...]; wh = gwh[...]
    seq0 = s0_ref[0]
    seq10 = s10_ref[0]

    hidden = jnp.zeros((B * V, H), jnp.float32)
    pred = None
    for s in range(NSTEP):
        ins = seq0 if s == 0 else (seq10 if s == PRED_STEPS else pred)
        rs = jnp.dot(relcat, hidden, preferred_element_type=jnp.float32)
        pre = jnp.concatenate([rs[:BE], rs[BE:]], axis=1)
        m1 = jnp.tanh(jnp.dot(pre, w1cat, preferred_element_type=jnp.float32) + b1cat)
        msgs = jnp.zeros((BE, H), jnp.float32)
        for k in range(3):
            mk = jnp.tanh(jnp.dot(m1[:, k * H:(k + 1) * H], w2s[k],
                                  preferred_element_type=jnp.float32) + b2s[k])
            msgs = msgs + (mk * masks[k]) / 3.0
        agg = jnp.dot(relrt, msgs, preferred_element_type=jnp.float32) / 3.0
        gin = jnp.dot(ins, wi, preferred_element_type=jnp.float32) + bi
        gh = jnp.dot(agg, wh, preferred_element_type=jnp.float32)
        r = jax.nn.sigmoid(gin[:, :H] + gh[:, :H])
        i = jax.nn.sigmoid(gin[:, H:2 * H] + gh[:, H:2 * H])
        nn = jnp.tanh(gin[:, 2 * H:] + r * gh[:, 2 * H:])
        hidden = (1.0 - i) * nn + i * hidden
        p = jax.nn.relu(jnp.dot(hidden, o1w[...], preferred_element_type=jnp.float32) + o1b[...])
        p = jax.nn.relu(jnp.dot(p, o2w[...], preferred_element_type=jnp.float32) + o2b[...])
        p = jnp.dot(p, o3w[...], preferred_element_type=jnp.float32) + o3b[...]
        pred = ins + p
        preds_ref[0, :, s, :] = pred


def _row(b):
    return b.reshape(1, -1)


def kernel(data, params, rel_rec, rel_send):
    p = params
    x = jnp.transpose(data, (0, 3, 1, 2)).reshape(G, B * V, N_IN_ENC)
    seq = jnp.transpose(data, (0, 2, 3, 1))
    s0 = seq[:, 0].reshape(G, B * V, C)
    s10 = seq[:, PRED_STEPS].reshape(G, B * V, C)
    u = jax.random.uniform(jax.random.key(42), (N, E, K),
                           dtype=jnp.float32).reshape(G, B * E, K)
    eyeb = jnp.eye(B, dtype=jnp.float32)
    relcat = jnp.concatenate([jnp.kron(eyeb, rel_rec),
                              jnp.kron(eyeb, rel_send)], axis=0)
    relrt = jnp.kron(eyeb, jnp.transpose(rel_rec))

    weights = []
    for name in ('mlp1', 'mlp2', 'mlp3', 'mlp4'):
        weights += [p[name + '_fc1_w'], _row(p[name + '_fc1_b']),
                    p[name + '_fc2_w'], _row(p[name + '_fc2_b']),
                    jnp.stack([p[name + '_bn_g'], p[name + '_bn_b']])]
    weights += [p['fc_out_w'], _row(p['fc_out_b'])]
    weights += [jnp.concatenate([p['msg1_%d_w' % k] for k in (1, 2, 3)], axis=1),
                jnp.concatenate([_row(p['msg1_%d_b' % k]) for k in (1, 2, 3)], axis=1)]
    for k in (1, 2, 3):
        weights += [p['msg2_%d_w' % k], _row(p['msg2_%d_b' % k])]
    weights += [jnp.concatenate([p['input_r_w'], p['input_i_w'], p['input_n_w']], axis=1),
                jnp.concatenate([_row(p['input_r_b']), _row(p['input_i_b']),
                                 _row(p['input_n_b'])], axis=1),
                jnp.concatenate([p['hidden_r_w'], p['hidden_i_w'], p['hidden_n_w']], axis=1)]
    weights += [p['out_fc1_w'], _row(p['out_fc1_b']),
                p['out_fc2_w'], _row(p['out_fc2_b']),
                p['out_fc3_w'], _row(p['out_fc3_b'])]

    def wspec(w):
        nd = w.ndim
        return pl.BlockSpec(w.shape, lambda i, _nd=nd: (0,) * _nd)

    in_specs = [
        pl.BlockSpec((1, B * V, N_IN_ENC), lambda i: (i, 0, 0)),
        pl.BlockSpec((1, B * V, C), lambda i: (i, 0, 0)),
        pl.BlockSpec((1, B * V, C), lambda i: (i, 0, 0)),
        pl.BlockSpec((1, B * E, K), lambda i: (i, 0, 0)),
        pl.BlockSpec((2 * B * E, B * V), lambda i: (0, 0)),
        pl.BlockSpec((B * V, B * E), lambda i: (0, 0)),
    ] + [wspec(w) for w in weights]

    out_specs = [
        pl.BlockSpec((1, B * V, NSTEP, C), lambda i: (i, 0, 0, 0)),
        pl.BlockSpec((1, B * E, K), lambda i: (i, 0, 0)),
    ]
    out_shape = [
        jax.ShapeDtypeStruct((G, B * V, NSTEP, C), jnp.float32),
        jax.ShapeDtypeStruct((G, B * E, K), jnp.float32),
    ]

    preds, prob = pl.pallas_call(
        _fwd_body,
        grid=(G,),
        in_specs=in_specs,
        out_specs=out_specs,
        out_shape=out_shape,
    )(x, s0, s10, u, relcat, relrt, *weights)
    return preds.reshape(N, V, NSTEP, C), prob.reshape(N, E, K)
